# baseline (speedup 1.0000x reference)
import jax
import jax.numpy as jnp
from jax.experimental import pallas as pl
from jax.experimental.pallas import tpu as pltpu

B = 160000
D = 128
C = 4
KP = 128
S = C * KP
R = 2000
NB = B // R


def _mask(a, c):
    ac = jnp.broadcast_to(a[:, c:c + 1], (a.shape[0], KP))
    k = jax.lax.broadcasted_iota(jnp.int32, (a.shape[0], KP), 1)
    return ac == k


def _onehot_bf16(a, c):
    ac = jnp.broadcast_to(a[:, c:c + 1], (a.shape[0], KP)).astype(jnp.int16)
    k = jax.lax.broadcasted_iota(jnp.int16, (a.shape[0], KP), 1)
    return jnp.where(ac == k, jnp.bfloat16(1.0), jnp.bfloat16(0.0))


def _row_of(col, eyef):
    return jax.lax.dot_general(col, eyef, (((0,), (0,)), ((), ())),
                               preferred_element_type=jnp.float32)


def _safe_sqrt(sq):
    pos = sq > 0.0
    return jnp.where(pos, jnp.sqrt(jnp.where(pos, sq, 1.0)), 0.0)


def _stats_finalize(sums_ref, ccol_ref, intra_ref, e_ref):
    eye = (jax.lax.broadcasted_iota(jnp.int32, (KP, KP), 0)
           == jax.lax.broadcasted_iota(jnp.int32, (KP, KP), 1))
    upper = (jax.lax.broadcasted_iota(jnp.int32, (KP, KP), 0)
             < jax.lax.broadcasted_iota(jnp.int32, (KP, KP), 1))
    eyef = eye.astype(jnp.float32)
    for c in range(C):
        cnt_col = ccol_ref[c * KP:(c + 1) * KP, :]
        cnt_row = _row_of(cnt_col, eyef)
        real = jax.lax.broadcasted_iota(jnp.int32, (1, KP), 1) < 100
        total = jnp.sum(cnt_row)
        probs = cnt_row / total + 1e-10
        H = -jnp.sum(jnp.where(real, probs * jnp.log(probs), 0.0))
        safe = jnp.maximum(cnt_row, 1.0)
        intra_mean = intra_ref[c:c + 1, :] / safe
        valid = cnt_row > 1.0
        n_valid = jnp.sum(valid.astype(jnp.float32))
        h_a = jnp.where(n_valid > 0.0,
                        jnp.sum(jnp.where(valid, intra_mean, 0.0))
                        / jnp.maximum(n_valid, 1.0), 0.0)
        cent = sums_ref[c * KP:(c + 1) * KP, :] / jnp.maximum(cnt_col, 1.0)
        G = jax.lax.dot_general(cent, cent, (((1,), (1,)), ((), ())),
                                preferred_element_type=jnp.float32,
                                precision=jax.lax.Precision.HIGHEST)
        csq_col = jnp.sum(G * eyef, axis=1, keepdims=True)
        csq_row = jnp.sum(G * eyef, axis=0, keepdims=True)
        inter = _safe_sqrt(csq_col + csq_row - 2.0 * G)
        pair_mask = upper & (cnt_col > 0.0) & (cnt_row > 0.0)
        n_pairs = jnp.sum(pair_mask.astype(jnp.float32))
        h_r = jnp.where(n_pairs > 0.0,
                        jnp.sum(jnp.where(pair_mask, inter, 0.0))
                        / jnp.maximum(n_pairs, 1.0), 0.0)
        min_intra = jnp.where(n_valid > 0.0,
                              jnp.min(jnp.where(valid, intra_mean, jnp.inf)),
                              0.0)
        max_inter = jnp.where(n_pairs > 0.0,
                              jnp.max(jnp.where(pair_mask, inter, -jnp.inf)),
                              0.0)
        delta_gamma = max_inter - min_intra
        few = jnp.sum((cnt_row > 0.0).astype(jnp.float32)) <= 1.0
        zero = jnp.float32(0.0)
        h_a = jnp.where(few, zero, h_a)
        h_r = jnp.where(few, zero, h_r)
        delta_gamma = jnp.where(few, zero, delta_gamma)
        row = jnp.concatenate(
            [H.reshape(1, 1), h_a.reshape(1, 1),
             h_r.reshape(1, 1), delta_gamma.reshape(1, 1)], axis=1)
        e_ref[c:c + 1, :] = row


def _fused_body(x_ref, a_ref, sums_ref, ccol_ref, intra_ref, e_ref, cent_ref):
    p = pl.program_id(0)
    i = pl.program_id(1)

    @pl.when((p == 0) & (i == 0))
    def _():
        sums_ref[...] = jnp.zeros_like(sums_ref)
        ccol_ref[...] = jnp.zeros_like(ccol_ref)

    @pl.when(p == 0)
    def _():
        x = x_ref[...]
        a = a_ref[...]
        ohs = [_onehot_bf16(a, c) for c in range(C)]
        oh_all = jnp.concatenate(ohs, axis=1)
        ones_col = jnp.ones((x.shape[0], 8), dtype=jnp.bfloat16)
        xb = jnp.concatenate([x.astype(jnp.bfloat16), ones_col], axis=1)
        res = jax.lax.dot_general(
            oh_all, xb, (((0,), (0,)), ((), ())),
            preferred_element_type=jnp.float32)
        sums_ref[...] += res[:, :D]
        ccol_ref[...] += res[:, D:D + 1]

    @pl.when((p == 1) & (i == 0))
    def _():
        cent = sums_ref[...] / jnp.maximum(ccol_ref[...], 1.0)
        cent_ref[...] = cent.astype(jnp.bfloat16)
        intra_ref[...] = jnp.zeros_like(intra_ref)

    @pl.when(p == 1)
    def _():
        xb = x_ref[...].astype(jnp.bfloat16)
        a = a_ref[...]
        for c in range(C):
            oh = _onehot_bf16(a, c)
            cg = jax.lax.dot_general(
                oh, cent_ref[c * KP:(c + 1) * KP, :], (((1,), (0,)), ((), ())),
                preferred_element_type=jnp.float32)
            db = xb - cg.astype(jnp.bfloat16)
            q = db * db
            s = jnp.sum(q, axis=1, keepdims=True, dtype=jnp.float32)
            d = _safe_sqrt(s)
            m = _mask(a, c)
            dsel = jnp.where(m, jnp.broadcast_to(d, m.shape), 0.0)
            intra_ref[c:c + 1, :] += jnp.sum(dsel, axis=0, keepdims=True)

    @pl.when((p == 1) & (i == NB - 1))
    def _():
        _stats_finalize(sums_ref, ccol_ref, intra_ref, e_ref)


def kernel(features, cluster_assignments):
    assert features.shape == (B, D)
    assert cluster_assignments.shape == (B, C)

    _, _, _, e = pl.pallas_call(
        _fused_body,
        grid=(2, NB),
        in_specs=[
            pl.BlockSpec((R, D), lambda p, i: (i, 0)),
            pl.BlockSpec((R, C), lambda p, i: (i, 0)),
        ],
        out_specs=[
            pl.BlockSpec((S, D), lambda p, i: (0, 0)),
            pl.BlockSpec((S, 1), lambda p, i: (0, 0)),
            pl.BlockSpec((C, KP), lambda p, i: (0, 0)),
            pl.BlockSpec((C, C), lambda p, i: (0, 0)),
        ],
        out_shape=[
            jax.ShapeDtypeStruct((S, D), jnp.float32),
            jax.ShapeDtypeStruct((S, 1), jnp.float32),
            jax.ShapeDtypeStruct((C, KP), jnp.float32),
            jax.ShapeDtypeStruct((C, C), jnp.float32),
        ],
        scratch_shapes=[pltpu.VMEM((S, D), jnp.bfloat16)],
    )(features, cluster_assignments)

    return jnp.broadcast_to(e[None, :, :], (B, C, C))


# speedup vs baseline: 1.3314x; 1.3314x over previous
import jax
import jax.numpy as jnp
from jax.experimental import pallas as pl
from jax.experimental.pallas import tpu as pltpu

B = 160000
D = 128
C = 4
KP = 128
S = C * KP
R = 4000
NB = B // R


def _mask(a, c):
    ac = jnp.broadcast_to(a[:, c:c + 1], (a.shape[0], KP))
    k = jax.lax.broadcasted_iota(jnp.int32, (a.shape[0], KP), 1)
    return ac == k


def _onehot_bf16(a, c):
    ac = jnp.broadcast_to(a[:, c:c + 1], (a.shape[0], KP)).astype(jnp.int16)
    k = jax.lax.broadcasted_iota(jnp.int16, (a.shape[0], KP), 1)
    return jnp.where(ac == k, jnp.bfloat16(1.0), jnp.bfloat16(0.0))


def _row_of(col, eyef):
    return jax.lax.dot_general(col, eyef, (((0,), (0,)), ((), ())),
                               preferred_element_type=jnp.float32)


def _safe_sqrt(sq):
    pos = sq > 0.0
    return jnp.where(pos, jnp.sqrt(jnp.where(pos, sq, 1.0)), 0.0)


def _stats_finalize(sums_ref, ccol_ref, intra_ref, e_ref):
    eye = (jax.lax.broadcasted_iota(jnp.int32, (KP, KP), 0)
           == jax.lax.broadcasted_iota(jnp.int32, (KP, KP), 1))
    upper = (jax.lax.broadcasted_iota(jnp.int32, (KP, KP), 0)
             < jax.lax.broadcasted_iota(jnp.int32, (KP, KP), 1))
    eyef = eye.astype(jnp.float32)
    for c in range(C):
        cnt_col = ccol_ref[c * KP:(c + 1) * KP, :]
        cnt_row = _row_of(cnt_col, eyef)
        real = jax.lax.broadcasted_iota(jnp.int32, (1, KP), 1) < 100
        total = jnp.sum(cnt_row)
        probs = cnt_row / total + 1e-10
        H = -jnp.sum(jnp.where(real, probs * jnp.log(probs), 0.0))
        safe = jnp.maximum(cnt_row, 1.0)
        intra_mean = intra_ref[c:c + 1, :] / safe
        valid = cnt_row > 1.0
        n_valid = jnp.sum(valid.astype(jnp.float32))
        h_a = jnp.where(n_valid > 0.0,
                        jnp.sum(jnp.where(valid, intra_mean, 0.0))
                        / jnp.maximum(n_valid, 1.0), 0.0)
        cent = sums_ref[c * KP:(c + 1) * KP, :] / jnp.maximum(cnt_col, 1.0)
        G = jax.lax.dot_general(cent, cent, (((1,), (1,)), ((), ())),
                                preferred_element_type=jnp.float32,
                                precision=jax.lax.Precision.HIGHEST)
        csq_col = jnp.sum(G * eyef, axis=1, keepdims=True)
        csq_row = jnp.sum(G * eyef, axis=0, keepdims=True)
        inter = _safe_sqrt(csq_col + csq_row - 2.0 * G)
        pair_mask = upper & (cnt_col > 0.0) & (cnt_row > 0.0)
        n_pairs = jnp.sum(pair_mask.astype(jnp.float32))
        h_r = jnp.where(n_pairs > 0.0,
                        jnp.sum(jnp.where(pair_mask, inter, 0.0))
                        / jnp.maximum(n_pairs, 1.0), 0.0)
        min_intra = jnp.where(n_valid > 0.0,
                              jnp.min(jnp.where(valid, intra_mean, jnp.inf)),
                              0.0)
        max_inter = jnp.where(n_pairs > 0.0,
                              jnp.max(jnp.where(pair_mask, inter, -jnp.inf)),
                              0.0)
        delta_gamma = max_inter - min_intra
        few = jnp.sum((cnt_row > 0.0).astype(jnp.float32)) <= 1.0
        zero = jnp.float32(0.0)
        h_a = jnp.where(few, zero, h_a)
        h_r = jnp.where(few, zero, h_r)
        delta_gamma = jnp.where(few, zero, delta_gamma)
        row = jnp.concatenate(
            [H.reshape(1, 1), h_a.reshape(1, 1),
             h_r.reshape(1, 1), delta_gamma.reshape(1, 1)], axis=1)
        e_ref[c:c + 1, :] = row


def _fused_body(x_ref, a_ref, sums_ref, ccol_ref, intra_ref, e_ref, cent_ref):
    p = pl.program_id(0)
    i = pl.program_id(1)

    @pl.when((p == 0) & (i == 0))
    def _():
        sums_ref[...] = jnp.zeros_like(sums_ref)
        ccol_ref[...] = jnp.zeros_like(ccol_ref)

    @pl.when(p == 0)
    def _():
        x = x_ref[...]
        a = a_ref[...]
        ohs = [_onehot_bf16(a, c) for c in range(C)]
        oh_all = jnp.concatenate(ohs, axis=1)
        ones_col = jnp.ones((x.shape[0], 8), dtype=jnp.bfloat16)
        xb = jnp.concatenate([x.astype(jnp.bfloat16), ones_col], axis=1)
        res = jax.lax.dot_general(
            oh_all, xb, (((0,), (0,)), ((), ())),
            preferred_element_type=jnp.float32)
        sums_ref[...] += res[:, :D]
        ccol_ref[...] += res[:, D:D + 1]

    @pl.when((p == 1) & (i == 0))
    def _():
        cent = sums_ref[...] / jnp.maximum(ccol_ref[...], 1.0)
        cent_ref[...] = cent.astype(jnp.bfloat16)
        intra_ref[...] = jnp.zeros_like(intra_ref)

    @pl.when(p == 1)
    def _():
        xb = x_ref[...].astype(jnp.bfloat16)
        a = a_ref[...]
        for c in range(C):
            oh = _onehot_bf16(a, c)
            cg = jax.lax.dot_general(
                oh, cent_ref[c * KP:(c + 1) * KP, :], (((1,), (0,)), ((), ())),
                preferred_element_type=jnp.float32)
            db = xb - cg.astype(jnp.bfloat16)
            q = db * db
            s = jnp.sum(q, axis=1, keepdims=True, dtype=jnp.float32)
            d = _safe_sqrt(s)
            m = _mask(a, c)
            dsel = jnp.where(m, jnp.broadcast_to(d, m.shape), 0.0)
            intra_ref[c:c + 1, :] += jnp.sum(dsel, axis=0, keepdims=True)

    @pl.when((p == 1) & (i == NB - 1))
    def _():
        _stats_finalize(sums_ref, ccol_ref, intra_ref, e_ref)


def kernel(features, cluster_assignments):
    assert features.shape == (B, D)
    assert cluster_assignments.shape == (B, C)

    _, _, _, e = pl.pallas_call(
        _fused_body,
        grid=(2, NB),
        in_specs=[
            pl.BlockSpec((R, D), lambda p, i: (i, 0)),
            pl.BlockSpec((R, C), lambda p, i: (i, 0)),
        ],
        out_specs=[
            pl.BlockSpec((S, D), lambda p, i: (0, 0)),
            pl.BlockSpec((S, 1), lambda p, i: (0, 0)),
            pl.BlockSpec((C, KP), lambda p, i: (0, 0)),
            pl.BlockSpec((C, C), lambda p, i: (0, 0)),
        ],
        out_shape=[
            jax.ShapeDtypeStruct((S, D), jnp.float32),
            jax.ShapeDtypeStruct((S, 1), jnp.float32),
            jax.ShapeDtypeStruct((C, KP), jnp.float32),
            jax.ShapeDtypeStruct((C, C), jnp.float32),
        ],
        scratch_shapes=[pltpu.VMEM((S, D), jnp.bfloat16)],
    )(features, cluster_assignments)

    return jnp.broadcast_to(e[None, :, :], (B, C, C))
